# baseline (speedup 1.0000x reference)
import functools

import jax
import jax.numpy as jnp
from jax import lax
from jax.experimental import pallas as pl
from jax.experimental.pallas import tpu as pltpu
from jax.experimental.pallas import tpu_sc as plsc

NC = 2
NS = 16
L = 16
NW = NC * NS

B = 256
F = 4 * 64 * 64
N = B * F
NT = 1000
TROW = 8 * B
NG = B // L

F_TC = 12288
F_SC = F - F_TC
N_SC = F_SC * B
ELEMS_W = N_SC // NW
CHUNK = 4 * TROW
NCHUNK = ELEMS_W // CHUNK
BR = 1024


def _sc_body(x_hbm, ts_hbm, n_hbm, sa_hbm, so_hbm, out_hbm,
             sa_v, so_v, ts_v, xb0, xb1, nb0, nb1, ob0, ob1,
             in_sem0, in_sem1, out_sem0, out_sem1):
    c = lax.axis_index("c")
    s = lax.axis_index("s")
    wid = s * NC + c
    base = F_TC * B + wid * ELEMS_W

    xbufs = (xb0, xb1)
    nbufs = (nb0, nb1)
    obufs = (ob0, ob1)
    in_sems = (in_sem0, in_sem1)
    out_sems = (out_sem0, out_sem1)

    pltpu.sync_copy(sa_hbm, sa_v)
    pltpu.sync_copy(so_hbm, so_v)
    pltpu.sync_copy(ts_hbm, ts_v)

    sa_gs = []
    so_gs = []
    for g in range(NG):
        tv = ts_v[pl.ds(g * L, L)]
        sa_gs.append(plsc.load_gather(sa_v, [tv]))
        so_gs.append(plsc.load_gather(so_v, [tv]))

    pltpu.async_copy(x_hbm.at[pl.ds(base, CHUNK)], xbufs[0], in_sems[0])
    pltpu.async_copy(n_hbm.at[pl.ds(base, CHUNK)], nbufs[0], in_sems[0])

    for j in range(NCHUNK):
        slot = j % 2
        nxt = (j + 1) % 2
        off = base + j * CHUNK
        if j + 1 < NCHUNK:
            on = off + CHUNK
            pltpu.async_copy(x_hbm.at[pl.ds(on, CHUNK)], xbufs[nxt],
                             in_sems[nxt])
            pltpu.async_copy(n_hbm.at[pl.ds(on, CHUNK)], nbufs[nxt],
                             in_sems[nxt])

        pltpu.make_async_copy(x_hbm.at[pl.ds(off, CHUNK)], xbufs[slot],
                              in_sems[slot]).wait()
        pltpu.make_async_copy(n_hbm.at[pl.ds(off, CHUNK)], nbufs[slot],
                              in_sems[slot]).wait()
        if j >= 2:
            op = base + (j - 2) * CHUNK - F_TC * B
            pltpu.make_async_copy(obufs[slot], out_hbm.at[pl.ds(op, CHUNK)],
                                  out_sems[slot]).wait()

        xs = xbufs[slot]
        ns = nbufs[slot]
        os_ = obufs[slot]

        @plsc.parallel_loop(0, CHUNK // B, 1, unroll=2)
        def _(q):
            o_base = (q >> 3) * TROW + (q & 7) * 128
            for tcol in range(2):
                for g8 in range(8):
                    o = o_base + tcol * 1024 + g8 * L
                    g = tcol * 8 + g8
                    xv = xs[pl.ds(o, L)]
                    nv = ns[pl.ds(o, L)]
                    os_[pl.ds(o, L)] = sa_gs[g] * xv + so_gs[g] * nv

        pltpu.async_copy(obufs[slot],
                         out_hbm.at[pl.ds(off - F_TC * B, CHUNK)],
                         out_sems[slot])

    for j in (NCHUNK - 2, NCHUNK - 1):
        op = base + j * CHUNK - F_TC * B
        pltpu.make_async_copy(obufs[j % 2], out_hbm.at[pl.ds(op, CHUNK)],
                              out_sems[j % 2]).wait()


def _tc_body(ts_ref, tab_ref, x_ref, n_ref, out_ref, coef_ref):
    @pl.when(pl.program_id(0) == 0)
    def _():
        iota_v = lax.broadcasted_iota(jnp.int32, (1024, B), 0)
        onehot = iota_v == ts_ref[...]
        sa_col = tab_ref[:, 0:1]
        so_col = tab_ref[:, 1:2]
        zero = jnp.zeros((1024, B), jnp.float32)
        coef_ref[0:1, :] = jnp.sum(jnp.where(onehot, sa_col, zero), axis=0,
                                   keepdims=True)
        coef_ref[1:2, :] = jnp.sum(jnp.where(onehot, so_col, zero), axis=0,
                                   keepdims=True)

    sa_row = coef_ref[0:1, :]
    so_row = coef_ref[1:2, :]
    out_ref[...] = sa_row * x_ref[...] + so_row * n_ref[...]


@jax.jit
def kernel(x_start, timesteps, noise, sqrt_alpha_cumprod,
           sqrt_one_minus_alpha_cumprod):
    xt = x_start.reshape(B, F).T
    nt = noise.reshape(B, F).T

    def to_flat(t2d):
        return t2d.reshape(F // 8, 8, 2, 128).transpose(0, 2, 1, 3).reshape(N)

    xf = to_flat(xt)
    nf = to_flat(nt)
    ts2 = timesteps.reshape(1, B)
    tabT = jnp.pad(
        jnp.stack([sqrt_alpha_cumprod, sqrt_one_minus_alpha_cumprod], axis=1),
        ((0, 1024 - NT), (0, 0)))

    sc_k = functools.partial(
        pl.kernel,
        out_type=jax.ShapeDtypeStruct((N_SC,), jnp.float32),
        mesh=plsc.VectorSubcoreMesh(core_axis_name="c", subcore_axis_name="s"),
        compiler_params=pltpu.CompilerParams(needs_layout_passes=False),
        scratch_types=[
            pltpu.VMEM((1024,), jnp.float32),
            pltpu.VMEM((1024,), jnp.float32),
            pltpu.VMEM((B,), jnp.int32),
            pltpu.VMEM((CHUNK,), jnp.float32),
            pltpu.VMEM((CHUNK,), jnp.float32),
            pltpu.VMEM((CHUNK,), jnp.float32),
            pltpu.VMEM((CHUNK,), jnp.float32),
            pltpu.VMEM((CHUNK,), jnp.float32),
            pltpu.VMEM((CHUNK,), jnp.float32),
            pltpu.SemaphoreType.DMA,
            pltpu.SemaphoreType.DMA,
            pltpu.SemaphoreType.DMA,
            pltpu.SemaphoreType.DMA,
        ],
    )(_sc_body)
    sa_p = jnp.pad(sqrt_alpha_cumprod, (0, 1024 - NT))
    so_p = jnp.pad(sqrt_one_minus_alpha_cumprod, (0, 1024 - NT))
    sc_out = sc_k(xf, timesteps, nf, sa_p, so_p)

    tc_out = pl.pallas_call(
        _tc_body,
        grid=(F_TC // BR,),
        in_specs=[
            pl.BlockSpec((1, B), lambda i: (0, 0)),
            pl.BlockSpec((1024, 2), lambda i: (0, 0)),
            pl.BlockSpec((BR, B), lambda i: (i, 0)),
            pl.BlockSpec((BR, B), lambda i: (i, 0)),
        ],
        out_specs=pl.BlockSpec((BR, B), lambda i: (i, 0)),
        out_shape=jax.ShapeDtypeStruct((F, B), jnp.float32),
        scratch_shapes=[pltpu.VMEM((2, B), jnp.float32)],
        compiler_params=pltpu.CompilerParams(
            dimension_semantics=("arbitrary",),
        ),
    )(ts2, tabT, xt, nt)

    sc_rows = (sc_out.reshape(F_SC // 8, 2, 8, 128)
               .transpose(0, 2, 1, 3).reshape(F_SC, B))
    out_t = lax.dynamic_update_slice(tc_out, sc_rows, (F_TC, 0))
    return out_t.T.reshape(x_start.shape)


# speedup vs baseline: 1.0288x; 1.0288x over previous
import functools

import jax
import jax.numpy as jnp
from jax import lax
from jax.experimental import pallas as pl
from jax.experimental.pallas import tpu as pltpu
from jax.experimental.pallas import tpu_sc as plsc

NC = 2
NS = 16
L = 16
NW = NC * NS

B = 256
F = 4 * 64 * 64
N = B * F
NT = 1000
TROW = 8 * B
NG = B // L

F_TC = 14336
F_SC = F - F_TC
N_SC = F_SC * B
ELEMS_W = N_SC // NW
CHUNK = 4 * TROW
NCHUNK = ELEMS_W // CHUNK
BR = 1024


def _sc_body(x_hbm, ts_hbm, n_hbm, sa_hbm, so_hbm, out_hbm,
             sa_v, so_v, ts_v, xb0, xb1, nb0, nb1, ob0, ob1,
             in_sem0, in_sem1, out_sem0, out_sem1):
    c = lax.axis_index("c")
    s = lax.axis_index("s")
    wid = s * NC + c
    base = F_TC * B + wid * ELEMS_W

    xbufs = (xb0, xb1)
    nbufs = (nb0, nb1)
    obufs = (ob0, ob1)
    in_sems = (in_sem0, in_sem1)
    out_sems = (out_sem0, out_sem1)

    pltpu.sync_copy(sa_hbm, sa_v)
    pltpu.sync_copy(so_hbm, so_v)
    pltpu.sync_copy(ts_hbm, ts_v)

    sa_gs = []
    so_gs = []
    for g in range(NG):
        tv = ts_v[pl.ds(g * L, L)]
        sa_gs.append(plsc.load_gather(sa_v, [tv]))
        so_gs.append(plsc.load_gather(so_v, [tv]))

    pltpu.async_copy(x_hbm.at[pl.ds(base, CHUNK)], xbufs[0], in_sems[0])
    pltpu.async_copy(n_hbm.at[pl.ds(base, CHUNK)], nbufs[0], in_sems[0])

    for j in range(NCHUNK):
        slot = j % 2
        nxt = (j + 1) % 2
        off = base + j * CHUNK
        if j + 1 < NCHUNK:
            on = off + CHUNK
            pltpu.async_copy(x_hbm.at[pl.ds(on, CHUNK)], xbufs[nxt],
                             in_sems[nxt])
            pltpu.async_copy(n_hbm.at[pl.ds(on, CHUNK)], nbufs[nxt],
                             in_sems[nxt])

        pltpu.make_async_copy(x_hbm.at[pl.ds(off, CHUNK)], xbufs[slot],
                              in_sems[slot]).wait()
        pltpu.make_async_copy(n_hbm.at[pl.ds(off, CHUNK)], nbufs[slot],
                              in_sems[slot]).wait()
        if j >= 2:
            op = base + (j - 2) * CHUNK - F_TC * B
            pltpu.make_async_copy(obufs[slot], out_hbm.at[pl.ds(op, CHUNK)],
                                  out_sems[slot]).wait()

        xs = xbufs[slot]
        ns = nbufs[slot]
        os_ = obufs[slot]

        @plsc.parallel_loop(0, CHUNK // B, 1, unroll=2)
        def _(q):
            o_base = (q >> 3) * TROW + (q & 7) * 128
            for tcol in range(2):
                for g8 in range(8):
                    o = o_base + tcol * 1024 + g8 * L
                    g = tcol * 8 + g8
                    xv = xs[pl.ds(o, L)]
                    nv = ns[pl.ds(o, L)]
                    os_[pl.ds(o, L)] = sa_gs[g] * xv + so_gs[g] * nv

        pltpu.async_copy(obufs[slot],
                         out_hbm.at[pl.ds(off - F_TC * B, CHUNK)],
                         out_sems[slot])

    for j in (NCHUNK - 2, NCHUNK - 1):
        op = base + j * CHUNK - F_TC * B
        pltpu.make_async_copy(obufs[j % 2], out_hbm.at[pl.ds(op, CHUNK)],
                              out_sems[j % 2]).wait()


def _tc_body(ts_ref, tab_ref, x_ref, n_ref, out_ref, coef_ref):
    @pl.when(pl.program_id(0) == 0)
    def _():
        iota_v = lax.broadcasted_iota(jnp.int32, (1024, B), 0)
        onehot = iota_v == ts_ref[...]
        sa_col = tab_ref[:, 0:1]
        so_col = tab_ref[:, 1:2]
        zero = jnp.zeros((1024, B), jnp.float32)
        coef_ref[0:1, :] = jnp.sum(jnp.where(onehot, sa_col, zero), axis=0,
                                   keepdims=True)
        coef_ref[1:2, :] = jnp.sum(jnp.where(onehot, so_col, zero), axis=0,
                                   keepdims=True)

    sa_row = coef_ref[0:1, :]
    so_row = coef_ref[1:2, :]
    out_ref[...] = sa_row * x_ref[...] + so_row * n_ref[...]


@jax.jit
def kernel(x_start, timesteps, noise, sqrt_alpha_cumprod,
           sqrt_one_minus_alpha_cumprod):
    xt = x_start.reshape(B, F).T
    nt = noise.reshape(B, F).T

    def to_flat(t2d):
        return t2d.reshape(F // 8, 8, 2, 128).transpose(0, 2, 1, 3).reshape(N)

    xf = to_flat(xt)
    nf = to_flat(nt)
    ts2 = timesteps.reshape(1, B)
    tabT = jnp.pad(
        jnp.stack([sqrt_alpha_cumprod, sqrt_one_minus_alpha_cumprod], axis=1),
        ((0, 1024 - NT), (0, 0)))

    sc_k = functools.partial(
        pl.kernel,
        out_type=jax.ShapeDtypeStruct((N_SC,), jnp.float32),
        mesh=plsc.VectorSubcoreMesh(core_axis_name="c", subcore_axis_name="s"),
        compiler_params=pltpu.CompilerParams(needs_layout_passes=False),
        scratch_types=[
            pltpu.VMEM((1024,), jnp.float32),
            pltpu.VMEM((1024,), jnp.float32),
            pltpu.VMEM((B,), jnp.int32),
            pltpu.VMEM((CHUNK,), jnp.float32),
            pltpu.VMEM((CHUNK,), jnp.float32),
            pltpu.VMEM((CHUNK,), jnp.float32),
            pltpu.VMEM((CHUNK,), jnp.float32),
            pltpu.VMEM((CHUNK,), jnp.float32),
            pltpu.VMEM((CHUNK,), jnp.float32),
            pltpu.SemaphoreType.DMA,
            pltpu.SemaphoreType.DMA,
            pltpu.SemaphoreType.DMA,
            pltpu.SemaphoreType.DMA,
        ],
    )(_sc_body)
    sa_p = jnp.pad(sqrt_alpha_cumprod, (0, 1024 - NT))
    so_p = jnp.pad(sqrt_one_minus_alpha_cumprod, (0, 1024 - NT))
    sc_out = sc_k(xf, timesteps, nf, sa_p, so_p)

    tc_out = pl.pallas_call(
        _tc_body,
        grid=(F_TC // BR,),
        in_specs=[
            pl.BlockSpec((1, B), lambda i: (0, 0)),
            pl.BlockSpec((1024, 2), lambda i: (0, 0)),
            pl.BlockSpec((BR, B), lambda i: (i, 0)),
            pl.BlockSpec((BR, B), lambda i: (i, 0)),
        ],
        out_specs=pl.BlockSpec((BR, B), lambda i: (i, 0)),
        out_shape=jax.ShapeDtypeStruct((F, B), jnp.float32),
        scratch_shapes=[pltpu.VMEM((2, B), jnp.float32)],
        compiler_params=pltpu.CompilerParams(
            dimension_semantics=("arbitrary",),
        ),
    )(ts2, tabT, xt, nt)

    sc_rows = (sc_out.reshape(F_SC // 8, 2, 8, 128)
               .transpose(0, 2, 1, 3).reshape(F_SC, B))
    out_t = lax.dynamic_update_slice(tc_out, sc_rows, (F_TC, 0))
    return out_t.T.reshape(x_start.shape)


# speedup vs baseline: 1.9824x; 1.9270x over previous
import jax
import jax.numpy as jnp
from jax import lax
from jax.experimental import pallas as pl
from jax.experimental.pallas import tpu as pltpu

B = 256
F = 4 * 64 * 64
NT = 1000
BR = 1024


def _body(ts_ref, tab_ref, x_ref, n_ref, out_ref, coef_ref):
    @pl.when(pl.program_id(0) == 0)
    def _():
        iota_v = lax.broadcasted_iota(jnp.int32, (1024, B), 0)
        onehot = iota_v == ts_ref[...]
        zero = jnp.zeros((1024, B), jnp.float32)
        coef_ref[0:1, :] = jnp.sum(
            jnp.where(onehot, tab_ref[:, 0:1], zero), axis=0, keepdims=True)
        coef_ref[1:2, :] = jnp.sum(
            jnp.where(onehot, tab_ref[:, 1:2], zero), axis=0, keepdims=True)

    sa_row = coef_ref[0:1, :]
    so_row = coef_ref[1:2, :]
    out_ref[...] = sa_row * x_ref[...] + so_row * n_ref[...]


@jax.jit
def kernel(x_start, timesteps, noise, sqrt_alpha_cumprod,
           sqrt_one_minus_alpha_cumprod):
    xt = x_start.reshape(B, F).T
    nt = noise.reshape(B, F).T
    ts2 = timesteps.reshape(1, B)
    tabT = jnp.pad(
        jnp.stack([sqrt_alpha_cumprod, sqrt_one_minus_alpha_cumprod], axis=1),
        ((0, 1024 - NT), (0, 0)))

    out_t = pl.pallas_call(
        _body,
        grid=(F // BR,),
        in_specs=[
            pl.BlockSpec((1, B), lambda i: (0, 0)),
            pl.BlockSpec((1024, 2), lambda i: (0, 0)),
            pl.BlockSpec((BR, B), lambda i: (i, 0)),
            pl.BlockSpec((BR, B), lambda i: (i, 0)),
        ],
        out_specs=pl.BlockSpec((BR, B), lambda i: (i, 0)),
        out_shape=jax.ShapeDtypeStruct((F, B), jnp.float32),
        scratch_shapes=[pltpu.VMEM((2, B), jnp.float32)],
        compiler_params=pltpu.CompilerParams(
            dimension_semantics=("arbitrary",),
        ),
    )(ts2, tabT, xt, nt)

    return out_t.T.reshape(x_start.shape)


# speedup vs baseline: 2.2054x; 1.1125x over previous
import jax
import jax.numpy as jnp
from jax import lax
from jax.experimental import pallas as pl
from jax.experimental.pallas import tpu as pltpu

B = 256
F = 4 * 64 * 64
NT = 1000
BR = 2048


def _body(ts_ref, tab_ref, x_ref, n_ref, out_ref, coef_ref):
    @pl.when(pl.program_id(0) == 0)
    def _():
        iota_v = lax.broadcasted_iota(jnp.int32, (1024, B), 0)
        onehot = iota_v == ts_ref[...]
        zero = jnp.zeros((1024, B), jnp.float32)
        coef_ref[0:1, :] = jnp.sum(
            jnp.where(onehot, tab_ref[:, 0:1], zero), axis=0, keepdims=True)
        coef_ref[1:2, :] = jnp.sum(
            jnp.where(onehot, tab_ref[:, 1:2], zero), axis=0, keepdims=True)

    sa_row = coef_ref[0:1, :]
    so_row = coef_ref[1:2, :]
    out_ref[...] = sa_row * x_ref[...] + so_row * n_ref[...]


@jax.jit
def kernel(x_start, timesteps, noise, sqrt_alpha_cumprod,
           sqrt_one_minus_alpha_cumprod):
    xt = x_start.reshape(B, F).T
    nt = noise.reshape(B, F).T
    ts2 = timesteps.reshape(1, B)
    tabT = jnp.pad(
        jnp.stack([sqrt_alpha_cumprod, sqrt_one_minus_alpha_cumprod], axis=1),
        ((0, 1024 - NT), (0, 0)))

    out_t = pl.pallas_call(
        _body,
        grid=(F // BR,),
        in_specs=[
            pl.BlockSpec((1, B), lambda i: (0, 0)),
            pl.BlockSpec((1024, 2), lambda i: (0, 0)),
            pl.BlockSpec((BR, B), lambda i: (i, 0)),
            pl.BlockSpec((BR, B), lambda i: (i, 0)),
        ],
        out_specs=pl.BlockSpec((BR, B), lambda i: (i, 0)),
        out_shape=jax.ShapeDtypeStruct((F, B), jnp.float32),
        scratch_shapes=[pltpu.VMEM((2, B), jnp.float32)],
        compiler_params=pltpu.CompilerParams(
            dimension_semantics=("arbitrary",),
        ),
    )(ts2, tabT, xt, nt)

    return out_t.T.reshape(x_start.shape)


# speedup vs baseline: 2.2187x; 1.0060x over previous
import jax
import jax.numpy as jnp
from jax import lax
from jax.experimental import pallas as pl
from jax.experimental.pallas import tpu as pltpu

B = 256
F = 4 * 64 * 64
NT = 1000
BR = 4096


def _body(ts_ref, tab_ref, x_ref, n_ref, out_ref, coef_ref):
    @pl.when(pl.program_id(0) == 0)
    def _():
        iota_v = lax.broadcasted_iota(jnp.int32, (1024, B), 0)
        onehot = iota_v == ts_ref[...]
        zero = jnp.zeros((1024, B), jnp.float32)
        coef_ref[0:1, :] = jnp.sum(
            jnp.where(onehot, tab_ref[:, 0:1], zero), axis=0, keepdims=True)
        coef_ref[1:2, :] = jnp.sum(
            jnp.where(onehot, tab_ref[:, 1:2], zero), axis=0, keepdims=True)

    sa_row = coef_ref[0:1, :]
    so_row = coef_ref[1:2, :]
    out_ref[...] = sa_row * x_ref[...] + so_row * n_ref[...]


@jax.jit
def kernel(x_start, timesteps, noise, sqrt_alpha_cumprod,
           sqrt_one_minus_alpha_cumprod):
    xt = x_start.reshape(B, F).T
    nt = noise.reshape(B, F).T
    ts2 = timesteps.reshape(1, B)
    tabT = jnp.pad(
        jnp.stack([sqrt_alpha_cumprod, sqrt_one_minus_alpha_cumprod], axis=1),
        ((0, 1024 - NT), (0, 0)))

    out_t = pl.pallas_call(
        _body,
        grid=(F // BR,),
        in_specs=[
            pl.BlockSpec((1, B), lambda i: (0, 0)),
            pl.BlockSpec((1024, 2), lambda i: (0, 0)),
            pl.BlockSpec((BR, B), lambda i: (i, 0)),
            pl.BlockSpec((BR, B), lambda i: (i, 0)),
        ],
        out_specs=pl.BlockSpec((BR, B), lambda i: (i, 0)),
        out_shape=jax.ShapeDtypeStruct((F, B), jnp.float32),
        scratch_shapes=[pltpu.VMEM((2, B), jnp.float32)],
        compiler_params=pltpu.CompilerParams(
            dimension_semantics=("arbitrary",),
        ),
    )(ts2, tabT, xt, nt)

    return out_t.T.reshape(x_start.shape)


# speedup vs baseline: 2.5382x; 1.1440x over previous
import jax
import jax.numpy as jnp
from jax import lax
from jax.experimental import pallas as pl
from jax.experimental.pallas import tpu as pltpu

B = 256
F = 4 * 64 * 64
NT = 1000
BR = 8192


def _body(ts_ref, tab_ref, x_ref, n_ref, out_ref, coef_ref):
    @pl.when(pl.program_id(0) == 0)
    def _():
        iota_v = lax.broadcasted_iota(jnp.int32, (1024, B), 0)
        onehot = iota_v == ts_ref[...]
        zero = jnp.zeros((1024, B), jnp.float32)
        coef_ref[0:1, :] = jnp.sum(
            jnp.where(onehot, tab_ref[:, 0:1], zero), axis=0, keepdims=True)
        coef_ref[1:2, :] = jnp.sum(
            jnp.where(onehot, tab_ref[:, 1:2], zero), axis=0, keepdims=True)

    sa_row = coef_ref[0:1, :]
    so_row = coef_ref[1:2, :]
    out_ref[...] = sa_row * x_ref[...] + so_row * n_ref[...]


@jax.jit
def kernel(x_start, timesteps, noise, sqrt_alpha_cumprod,
           sqrt_one_minus_alpha_cumprod):
    xt = x_start.reshape(B, F).T
    nt = noise.reshape(B, F).T
    ts2 = timesteps.reshape(1, B)
    tabT = jnp.pad(
        jnp.stack([sqrt_alpha_cumprod, sqrt_one_minus_alpha_cumprod], axis=1),
        ((0, 1024 - NT), (0, 0)))

    out_t = pl.pallas_call(
        _body,
        grid=(F // BR,),
        in_specs=[
            pl.BlockSpec((1, B), lambda i: (0, 0)),
            pl.BlockSpec((1024, 2), lambda i: (0, 0)),
            pl.BlockSpec((BR, B), lambda i: (i, 0)),
            pl.BlockSpec((BR, B), lambda i: (i, 0)),
        ],
        out_specs=pl.BlockSpec((BR, B), lambda i: (i, 0)),
        out_shape=jax.ShapeDtypeStruct((F, B), jnp.float32),
        scratch_shapes=[pltpu.VMEM((2, B), jnp.float32)],
        compiler_params=pltpu.CompilerParams(
            dimension_semantics=("arbitrary",),
        ),
    )(ts2, tabT, xt, nt)

    return out_t.T.reshape(x_start.shape)


# speedup vs baseline: 2.5436x; 1.0021x over previous
import jax
import jax.numpy as jnp
from jax import lax
from jax.experimental import pallas as pl
from jax.experimental.pallas import tpu as pltpu

B = 256
F = 4 * 64 * 64
NT = 1000
BR = 8192


def _body(ts_ref, tab_ref, x_ref, n_ref, out_ref, coef_ref):
    @pl.when(pl.program_id(0) == 0)
    def _():
        iota_v = lax.broadcasted_iota(jnp.int32, (NT, B), 0)
        onehot = iota_v == ts_ref[...]
        zero = jnp.zeros((NT, B), jnp.float32)
        coef_ref[0:1, :] = jnp.sum(
            jnp.where(onehot, tab_ref[:, 0:1], zero), axis=0, keepdims=True)
        coef_ref[1:2, :] = jnp.sum(
            jnp.where(onehot, tab_ref[:, 1:2], zero), axis=0, keepdims=True)

    sa_row = coef_ref[0:1, :]
    so_row = coef_ref[1:2, :]
    out_ref[...] = sa_row * x_ref[...] + so_row * n_ref[...]


@jax.jit
def kernel(x_start, timesteps, noise, sqrt_alpha_cumprod,
           sqrt_one_minus_alpha_cumprod):
    xt = x_start.reshape(B, F).T
    nt = noise.reshape(B, F).T
    ts2 = timesteps.reshape(1, B)
    tabT = jnp.stack([sqrt_alpha_cumprod, sqrt_one_minus_alpha_cumprod],
                     axis=1)

    out_t = pl.pallas_call(
        _body,
        grid=(F // BR,),
        in_specs=[
            pl.BlockSpec((1, B), lambda i: (0, 0)),
            pl.BlockSpec((NT, 2), lambda i: (0, 0)),
            pl.BlockSpec((BR, B), lambda i: (i, 0)),
            pl.BlockSpec((BR, B), lambda i: (i, 0)),
        ],
        out_specs=pl.BlockSpec((BR, B), lambda i: (i, 0)),
        out_shape=jax.ShapeDtypeStruct((F, B), jnp.float32),
        scratch_shapes=[pltpu.VMEM((2, B), jnp.float32)],
        compiler_params=pltpu.CompilerParams(
            dimension_semantics=("arbitrary",),
        ),
    )(ts2, tabT, xt, nt)

    return out_t.T.reshape(x_start.shape)
